# baseline (speedup 1.0000x reference)
import functools

import jax
import jax.numpy as jnp
from jax import lax
from jax.experimental import pallas as pl
from jax.experimental.pallas import tpu as pltpu
from jax.experimental.pallas import tpu_sc as plsc

N_NODES = 10000
N_EDGES = 320000
D = 128
NUM_RELS = 50
BN_EPS = 1e-5

NC = 2
NS = 16
NW = NC * NS
EW = N_EDGES // NW
B = 80
NB = EW // B
AW = D + 16
NCHUNKS = N_NODES // B


def _dense_body(x_ref, wk_ref, bk_ref, wq_ref, bq_ref, wv_ref, bv_ref,
                ws_ref, bs_ref, wc_ref, ra_ref, rf_ref, wr_ref, br_ref,
                kv_ref, q_ref, s_ref, relw_ref, rout_ref):
    x = x_ref[...]

    def proj(w_ref, b_ref):
        return lax.dot_general(x, w_ref[...], (((1,), (1,)), ((), ())),
                               preferred_element_type=jnp.float32) + b_ref[...]

    kv_ref[:, :D] = proj(wk_ref, bk_ref).astype(jnp.bfloat16)
    kv_ref[:, D:] = proj(wv_ref, bv_ref).astype(jnp.bfloat16)
    q_ref[...] = proj(wq_ref, bq_ref).astype(jnp.bfloat16)
    s_ref[...] = proj(ws_ref, bs_ref)
    relw_ref[...] = lax.dot_general(wc_ref[...], ra_ref[...],
                                    (((1,), (0,)), ((), ())),
                                    preferred_element_type=jnp.float32
                                    ).astype(jnp.bfloat16)
    rout_ref[...] = lax.dot_general(rf_ref[...], wr_ref[...],
                                    (((1,), (1,)), ((), ())),
                                    preferred_element_type=jnp.float32) + br_ref[...]


_dense_call = pl.pallas_call(
    _dense_body,
    out_shape=[
        jax.ShapeDtypeStruct((N_NODES, 2 * D), jnp.bfloat16),
        jax.ShapeDtypeStruct((N_NODES, D), jnp.bfloat16),
        jax.ShapeDtypeStruct((N_NODES, D), jnp.float32),
        jax.ShapeDtypeStruct((NUM_RELS, D), jnp.bfloat16),
        jax.ShapeDtypeStruct((NUM_RELS - 1, D), jnp.float32),
    ],
)


def _edge_body(kv_hbm, q_hbm, relw_hbm, idx_hbm, outv_hbm, outd_hbm,
               idx_v, relw_v, kv_v, q_v, pv_v, pd_v, acc_sh, den_sh,
               sem_k, sem_q, sem_i, sem_s, sem_d):
    c = lax.axis_index("c")
    s = lax.axis_index("s")
    wid = s * NC + c
    DW = D // 2

    pltpu.sync_copy(relw_hbm, relw_v)

    z16 = jnp.zeros((16,), jnp.float32)

    def zero_row(r, _):
        for cc in range(D // 16):
            pv_v[r, pl.ds(cc * 16, 16)] = z16
        pd_v[r, pl.ds(0, 16)] = z16
        return 0

    lax.fori_loop(0, B, zero_row, 0)

    for i in range(-(-NCHUNKS // NS)):
        ch = s + i * NS

        @pl.when(ch < NCHUNKS)
        def _():
            pltpu.sync_copy(pv_v, acc_sh.at[pl.ds(ch * B, B)])
            pltpu.sync_copy(pd_v, den_sh.at[pl.ds(ch * B, B)])
    plsc.subcore_barrier()

    lanes = lax.iota(jnp.int32, 16)

    pltpu.sync_copy(idx_hbm.at[wid, 0], idx_v.at[0])
    pltpu.async_copy(idx_hbm.at[wid, lax.min(1, NB - 1)], idx_v.at[1], sem_i)
    pltpu.async_copy(kv_hbm.at[idx_v.at[0].at[0]], kv_v, sem_k)
    pltpu.async_copy(q_hbm.at[idx_v.at[0].at[1]], q_v, sem_q)

    def batch_body(b, _):
        m = lax.rem(b, 3)
        m1 = lax.rem(b + 1, 3)
        m2 = lax.rem(b + 2, 3)
        bn = lax.min(b + 1, NB - 1)
        bn2 = lax.min(b + 2, NB - 1)
        cur = idx_v.at[m]
        nxt = idx_v.at[m1]
        with jax.named_scope("ph_gwait"):
            pltpu.make_async_copy(kv_hbm.at[cur.at[0]], kv_v, sem_k).wait()
            pltpu.make_async_copy(q_hbm.at[cur.at[1]], q_v, sem_q).wait()

            @pl.when(b > 0)
            def _():
                pltpu.make_async_copy(pd_v, den_sh.at[cur.at[1]],
                                      sem_d).wait()
        ns_att = jax.named_scope("ph_att"); ns_att.__enter__()
        for g in range(B // 16):
            row = lanes + (g * 16)
            et = idx_v[m, 2, pl.ds(g * 16, 16)]

            def att_body(j, acc4):
                accs = list(acc4)
                for t in range(2):
                    jc = (jnp.full((16,), j, jnp.int32) + t + lanes) & (DW - 1)
                    kk = plsc.bitcast(plsc.load_gather(kv_v, [row, jc]),
                                      jnp.bfloat16)
                    ww = plsc.bitcast(plsc.load_gather(relw_v, [et, jc]),
                                      jnp.bfloat16)
                    qq = plsc.bitcast(plsc.load_gather(q_v, [row, jc]),
                                      jnp.bfloat16)
                    lo, hi = plsc.unpack(kk * ww * qq,
                                         format=plsc.PackFormat.INTERLEAVED)
                    accs[2 * t] = accs[2 * t] + lo
                    accs[2 * t + 1] = accs[2 * t + 1] + hi
                return tuple(accs)

            z = jnp.zeros((16,), jnp.float32)
            a0, a1, a2, a3 = plsc.parallel_loop(
                0, DW, 2, unroll=4, carry=(z, z, z, z))(att_body)
            att = (a0 + a1) + (a2 + a3)
            plsc.store_scatter(pd_v, [row, lanes], jnp.exp(att))
        ns_att.__exit__(None, None, None)
        with jax.named_scope("ph_issue"):
            pltpu.make_async_copy(idx_hbm.at[wid, bn], nxt, sem_i).wait()
            pltpu.async_copy(kv_hbm.at[nxt.at[0]], kv_v, sem_k)
            pltpu.async_copy(q_hbm.at[nxt.at[1]], q_v, sem_q)
        with jax.named_scope("ph_vwait"):
            @pl.when(b > 0)
            def _():
                pltpu.make_async_copy(pv_v, acc_sh.at[cur.at[1]],
                                      sem_s).wait()
        ns_pv = jax.named_scope("ph_pv"); ns_pv.__enter__()
        for g in range(B // 16):
            row = lanes + (g * 16)
            p = plsc.load_gather(pd_v, [row, lanes])

            @plsc.parallel_loop(0, DW, 2, unroll=2)
            def pv_body(j):
                for t in range(2):
                    jc = (jnp.full((16,), j, jnp.int32) + t + lanes) & (DW - 1)
                    vv = plsc.bitcast(plsc.load_gather(kv_v, [row, jc + DW]),
                                      jnp.bfloat16)
                    lo, hi = plsc.unpack(vv,
                                         format=plsc.PackFormat.INTERLEAVED)
                    plsc.store_scatter(pv_v, [row, 2 * jc], p * lo)
                    plsc.store_scatter(pv_v, [row, 2 * jc + 1], p * hi)
        ns_pv.__exit__(None, None, None)
        with jax.named_scope("ph_scat"):
            pltpu.async_copy(pv_v, acc_sh.at[cur.at[1]], sem_s, add=True)
            pltpu.async_copy(pd_v, den_sh.at[cur.at[1]], sem_d, add=True)
        with jax.named_scope("ph_ipf"):
            pltpu.async_copy(idx_hbm.at[wid, bn2], idx_v.at[m2], sem_i)
        return 0

    lax.fori_loop(0, NB, batch_body, 0)
    pltpu.make_async_copy(kv_hbm.at[idx_v.at[0].at[0]], kv_v, sem_k).wait()
    pltpu.make_async_copy(q_hbm.at[idx_v.at[0].at[1]], q_v, sem_q).wait()
    pltpu.make_async_copy(idx_hbm.at[wid, 0], idx_v.at[0], sem_i).wait()
    pltpu.make_async_copy(pv_v, acc_sh.at[idx_v.at[0].at[1]], sem_s).wait()
    pltpu.make_async_copy(pd_v, den_sh.at[idx_v.at[0].at[1]], sem_d).wait()
    plsc.subcore_barrier()

    for i in range(-(-NCHUNKS // NS)):
        ch = s + i * NS

        @pl.when(ch < NCHUNKS)
        def _():
            pltpu.sync_copy(acc_sh.at[pl.ds(ch * B, B)],
                            outv_hbm.at[c, pl.ds(ch * B, B)])
            pltpu.sync_copy(den_sh.at[pl.ds(ch * B, B)],
                            outd_hbm.at[c, pl.ds(ch * B, B)])


_edge_call = functools.partial(
    pl.kernel,
    out_type=[
        jax.ShapeDtypeStruct((NC, N_NODES, D), jnp.float32),
        jax.ShapeDtypeStruct((NC, N_NODES, 16), jnp.float32),
    ],
    mesh=plsc.VectorSubcoreMesh(core_axis_name="c", subcore_axis_name="s",
                                num_cores=NC, num_subcores=NS),
    compiler_params=pltpu.CompilerParams(use_tc_tiling_on_sc=False,
                                         needs_layout_passes=False),
    scratch_types=[
        pltpu.VMEM((3, 3, B), jnp.int32),
        pltpu.VMEM((NUM_RELS, D // 2), jnp.int32),
        pltpu.VMEM((B, D), jnp.int32),
        pltpu.VMEM((B, D // 2), jnp.int32),
        pltpu.VMEM((B, D), jnp.float32),
        pltpu.VMEM((B, 16), jnp.float32),
        pltpu.VMEM_SHARED((N_NODES, D), jnp.float32),
        pltpu.VMEM_SHARED((N_NODES, 16), jnp.float32),
        pltpu.SemaphoreType.DMA,
        pltpu.SemaphoreType.DMA,
        pltpu.SemaphoreType.DMA,
        pltpu.SemaphoreType.DMA,
        pltpu.SemaphoreType.DMA,
    ],
)(_edge_body)


def _fin_body(acc_ref, den_ref, s_ref, alpha_ref, g_ref, b_ref, out_ref):
    num = acc_ref[0] + acc_ref[1]
    den = jnp.sum(den_ref[0] + den_ref[1], axis=1, keepdims=True)
    final = num / jnp.where(den > 0.0, den, 1.0)
    a = jax.nn.sigmoid(alpha_ref[0, 0])
    n_out = a * s_ref[...] + (1.0 - a) * final
    mean = jnp.mean(n_out, axis=0, keepdims=True)
    var = jnp.mean((n_out - mean) ** 2, axis=0, keepdims=True)
    out_ref[...] = jnp.tanh((n_out - mean) * lax.rsqrt(var + BN_EPS)
                            * g_ref[...] + b_ref[...])


_fin_call = pl.pallas_call(
    _fin_body,
    out_shape=jax.ShapeDtypeStruct((N_NODES, D), jnp.float32),
)


def kernel(n_in_feats, r_feats, edge_index, etype, norm,
           W_S_w, W_S_b, Wk_w, Wk_b, Wq_w, Wq_b, Wv_w, Wv_b,
           W_R_w, W_R_b, relation_att, w_comp, alpha, loop_rel,
           bn_gamma, bn_beta):
    del norm, loop_rel
    idx = jnp.concatenate([edge_index, etype[None]], axis=0)
    idx = idx.reshape(3, NW, NB, B).transpose(1, 2, 0, 3)

    kv, q, s, relw, r_out = _dense_call(
        n_in_feats, Wk_w, Wk_b.reshape(1, D), Wq_w, Wq_b.reshape(1, D),
        Wv_w, Wv_b.reshape(1, D), W_S_w, W_S_b.reshape(1, D),
        w_comp, relation_att, r_feats, W_R_w, W_R_b.reshape(1, D))

    def words(x):
        return lax.bitcast_convert_type(
            x.reshape(x.shape[0], x.shape[1] // 2, 2), jnp.int32)

    acc, den = _edge_call(words(kv), words(q), words(relw), idx)

    n_out = _fin_call(acc, den, s, alpha.reshape(1, 1),
                      bn_gamma.reshape(1, D), bn_beta.reshape(1, D))
    return n_out, r_out


# speedup vs baseline: 1.3599x; 1.3599x over previous
import functools

import jax
import jax.numpy as jnp
from jax import lax
from jax.experimental import pallas as pl
from jax.experimental.pallas import tpu as pltpu
from jax.experimental.pallas import tpu_sc as plsc

N_NODES = 10000
N_EDGES = 320000
D = 128
NUM_RELS = 50
BN_EPS = 1e-5

NC = 2
NS = 16
NW = NC * NS
EW = N_EDGES // NW
B = 80
NB = EW // B
AW = D + 16
NCHUNKS = N_NODES // B


def _dense_body(x_ref, wk_ref, bk_ref, wq_ref, bq_ref, wv_ref, bv_ref,
                ws_ref, bs_ref, wc_ref, ra_ref, rf_ref, wr_ref, br_ref,
                kv_ref, q_ref, s_ref, relw_ref, rout_ref):
    x = x_ref[...]

    def proj(w_ref, b_ref):
        return lax.dot_general(x, w_ref[...], (((1,), (1,)), ((), ())),
                               preferred_element_type=jnp.float32) + b_ref[...]

    def pack_halves(x):
        w = x.shape[1] // 2

        def rne(y):
            t = lax.bitcast_convert_type(y, jnp.int32)
            return t + 0x7FFF + ((t >> 16) & 1)

        lo = lax.shift_right_logical(rne(x[:, :w]), 16)
        hi = lax.bitwise_and(rne(x[:, w:]), jnp.int32(-65536))
        return lax.bitwise_or(hi, lo)

    kv_ref[:, :D // 2] = pack_halves(proj(wk_ref, bk_ref))
    kv_ref[:, D // 2:] = pack_halves(proj(wv_ref, bv_ref))
    q_ref[...] = pack_halves(proj(wq_ref, bq_ref))
    s_ref[...] = proj(ws_ref, bs_ref)
    relw_ref[...] = pack_halves(lax.dot_general(
        wc_ref[...], ra_ref[...], (((1,), (0,)), ((), ())),
        preferred_element_type=jnp.float32))
    rout_ref[...] = lax.dot_general(rf_ref[...], wr_ref[...],
                                    (((1,), (1,)), ((), ())),
                                    preferred_element_type=jnp.float32) + br_ref[...]


_dense_call = pl.pallas_call(
    _dense_body,
    out_shape=[
        jax.ShapeDtypeStruct((N_NODES, D), jnp.int32),
        jax.ShapeDtypeStruct((N_NODES, D // 2), jnp.int32),
        jax.ShapeDtypeStruct((N_NODES, D), jnp.float32),
        jax.ShapeDtypeStruct((NUM_RELS, D // 2), jnp.int32),
        jax.ShapeDtypeStruct((NUM_RELS - 1, D), jnp.float32),
    ],
)


def _edge_body(kv_hbm, q_hbm, relw_hbm, idx_hbm, outv_hbm, outd_hbm,
               idx_v, relw_v, kv_v, q_v, pv_v, pd_v, acc_sh, den_sh,
               sem_k, sem_q, sem_i, sem_s, sem_d):
    c = lax.axis_index("c")
    s = lax.axis_index("s")
    wid = s * NC + c
    DW = D // 2

    pltpu.sync_copy(relw_hbm, relw_v)

    z16 = jnp.zeros((16,), jnp.float32)

    def zero_row(r, _):
        for cc in range(D // 16):
            pv_v[r, pl.ds(cc * 16, 16)] = z16
        pd_v[r, pl.ds(0, 16)] = z16
        return 0

    lax.fori_loop(0, B, zero_row, 0)

    for i in range(-(-NCHUNKS // NS)):
        ch = s + i * NS

        @pl.when(ch < NCHUNKS)
        def _():
            pltpu.sync_copy(pv_v, acc_sh.at[pl.ds(ch * B, B)])
            pltpu.sync_copy(pd_v, den_sh.at[pl.ds(ch * B, B)])
    plsc.subcore_barrier()

    lanes = lax.iota(jnp.int32, 16)

    def idx_fetch(slot, batch, sync):
        for i in range(3):
            if sync:
                pltpu.sync_copy(idx_hbm.at[i, wid * NB + batch],
                                idx_v.at[slot, i])
            else:
                pltpu.async_copy(idx_hbm.at[i, wid * NB + batch],
                                 idx_v.at[slot, i], sem_i)

    idx_fetch(0, 0, True)
    idx_fetch(1, lax.min(1, NB - 1), False)
    pltpu.async_copy(kv_hbm.at[idx_v.at[0].at[0]], kv_v, sem_k)
    pltpu.async_copy(q_hbm.at[idx_v.at[0].at[1]], q_v, sem_q)

    def batch_body(b, _):
        m = lax.rem(b, 3)
        m1 = lax.rem(b + 1, 3)
        m2 = lax.rem(b + 2, 3)
        bn = lax.min(b + 1, NB - 1)
        bn2 = lax.min(b + 2, NB - 1)
        cur = idx_v.at[m]
        nxt = idx_v.at[m1]
        with jax.named_scope("ph_gwait"):
            pltpu.make_async_copy(kv_hbm.at[cur.at[0]], kv_v, sem_k).wait()
            pltpu.make_async_copy(q_hbm.at[cur.at[1]], q_v, sem_q).wait()

            @pl.when(b > 0)
            def _():
                pltpu.make_async_copy(pd_v, den_sh.at[cur.at[1]],
                                      sem_d).wait()
        ns_att = jax.named_scope("ph_att"); ns_att.__enter__()
        for g in range(B // 16):
            row = lanes + (g * 16)
            et = idx_v[m, 2, pl.ds(g * 16, 16)]

            def att_body(j, acc4):
                accs = list(acc4)
                for t in range(2):
                    jc = (jnp.full((16,), j, jnp.int32) + t + lanes) & (DW - 1)
                    kk = plsc.bitcast(plsc.load_gather(kv_v, [row, jc]),
                                      jnp.bfloat16)
                    ww = plsc.bitcast(plsc.load_gather(relw_v, [et, jc]),
                                      jnp.bfloat16)
                    qq = plsc.bitcast(plsc.load_gather(q_v, [row, jc]),
                                      jnp.bfloat16)
                    lo, hi = plsc.unpack(kk * ww * qq,
                                         format=plsc.PackFormat.INTERLEAVED)
                    accs[2 * t] = accs[2 * t] + lo
                    accs[2 * t + 1] = accs[2 * t + 1] + hi
                return tuple(accs)

            z = jnp.zeros((16,), jnp.float32)
            a0, a1, a2, a3 = plsc.parallel_loop(
                0, DW, 2, unroll=4, carry=(z, z, z, z))(att_body)
            att = (a0 + a1) + (a2 + a3)
            plsc.store_scatter(pd_v, [row, lanes], jnp.exp(att))
        ns_att.__exit__(None, None, None)
        with jax.named_scope("ph_issue"):
            for i in range(3):
                pltpu.make_async_copy(idx_hbm.at[i, wid * NB + bn],
                                      idx_v.at[m1, i], sem_i).wait()
            pltpu.async_copy(kv_hbm.at[nxt.at[0]], kv_v, sem_k)
            pltpu.async_copy(q_hbm.at[nxt.at[1]], q_v, sem_q)
        with jax.named_scope("ph_vwait"):
            @pl.when(b > 0)
            def _():
                pltpu.make_async_copy(pv_v, acc_sh.at[cur.at[1]],
                                      sem_s).wait()
        ns_pv = jax.named_scope("ph_pv"); ns_pv.__enter__()
        for g in range(B // 16):
            row = lanes + (g * 16)
            p = plsc.load_gather(pd_v, [row, lanes])

            @plsc.parallel_loop(0, DW, 2, unroll=2)
            def pv_body(j):
                for t in range(2):
                    jc = (jnp.full((16,), j, jnp.int32) + t + lanes) & (DW - 1)
                    vv = plsc.bitcast(plsc.load_gather(kv_v, [row, jc + DW]),
                                      jnp.bfloat16)
                    lo, hi = plsc.unpack(vv,
                                         format=plsc.PackFormat.INTERLEAVED)
                    plsc.store_scatter(pv_v, [row, jc], p * lo)
                    plsc.store_scatter(pv_v, [row, jc + DW], p * hi)
        ns_pv.__exit__(None, None, None)
        with jax.named_scope("ph_scat"):
            pltpu.async_copy(pv_v, acc_sh.at[cur.at[1]], sem_s, add=True)
            pltpu.async_copy(pd_v, den_sh.at[cur.at[1]], sem_d, add=True)
        with jax.named_scope("ph_ipf"):
            idx_fetch(m2, bn2, False)
        return 0

    lax.fori_loop(0, NB, batch_body, 0)
    pltpu.make_async_copy(kv_hbm.at[idx_v.at[0].at[0]], kv_v, sem_k).wait()
    pltpu.make_async_copy(q_hbm.at[idx_v.at[0].at[1]], q_v, sem_q).wait()
    for i in range(3):
        pltpu.make_async_copy(idx_hbm.at[i, wid * NB], idx_v.at[0, i],
                              sem_i).wait()
    pltpu.make_async_copy(pv_v, acc_sh.at[idx_v.at[0].at[1]], sem_s).wait()
    pltpu.make_async_copy(pd_v, den_sh.at[idx_v.at[0].at[1]], sem_d).wait()
    plsc.subcore_barrier()

    for i in range(-(-NCHUNKS // NS)):
        ch = s + i * NS

        @pl.when(ch < NCHUNKS)
        def _():
            pltpu.sync_copy(acc_sh.at[pl.ds(ch * B, B)],
                            outv_hbm.at[c, pl.ds(ch * B, B)])
            pltpu.sync_copy(den_sh.at[pl.ds(ch * B, B)],
                            outd_hbm.at[c, pl.ds(ch * B, B)])


_edge_call = functools.partial(
    pl.kernel,
    out_type=[
        jax.ShapeDtypeStruct((NC, N_NODES, D), jnp.float32),
        jax.ShapeDtypeStruct((NC, N_NODES, 16), jnp.float32),
    ],
    mesh=plsc.VectorSubcoreMesh(core_axis_name="c", subcore_axis_name="s",
                                num_cores=NC, num_subcores=NS),
    compiler_params=pltpu.CompilerParams(use_tc_tiling_on_sc=False,
                                         needs_layout_passes=False),
    scratch_types=[
        pltpu.VMEM((3, 3, B), jnp.int32),
        pltpu.VMEM((NUM_RELS, D // 2), jnp.int32),
        pltpu.VMEM((B, D), jnp.int32),
        pltpu.VMEM((B, D // 2), jnp.int32),
        pltpu.VMEM((B, D), jnp.float32),
        pltpu.VMEM((B, 16), jnp.float32),
        pltpu.VMEM_SHARED((N_NODES, D), jnp.float32),
        pltpu.VMEM_SHARED((N_NODES, 16), jnp.float32),
        pltpu.SemaphoreType.DMA,
        pltpu.SemaphoreType.DMA,
        pltpu.SemaphoreType.DMA,
        pltpu.SemaphoreType.DMA,
        pltpu.SemaphoreType.DMA,
    ],
)(_edge_body)


def _fin_body(acc_ref, den_ref, s_ref, alpha_ref, g_ref, b_ref, out_ref):
    num = acc_ref[0] + acc_ref[1]
    den = jnp.sum(den_ref[0] + den_ref[1], axis=1, keepdims=True)
    final = num / jnp.where(den > 0.0, den, 1.0)
    a = jax.nn.sigmoid(alpha_ref[0, 0])
    n_out = a * s_ref[...] + (1.0 - a) * final
    mean = jnp.mean(n_out, axis=0, keepdims=True)
    var = jnp.mean((n_out - mean) ** 2, axis=0, keepdims=True)
    out_ref[...] = jnp.tanh((n_out - mean) * lax.rsqrt(var + BN_EPS)
                            * g_ref[...] + b_ref[...])


_fin_call = pl.pallas_call(
    _fin_body,
    out_shape=jax.ShapeDtypeStruct((N_NODES, D), jnp.float32),
)


def kernel(n_in_feats, r_feats, edge_index, etype, norm,
           W_S_w, W_S_b, Wk_w, Wk_b, Wq_w, Wq_b, Wv_w, Wv_b,
           W_R_w, W_R_b, relation_att, w_comp, alpha, loop_rel,
           bn_gamma, bn_beta):
    del norm, loop_rel
    idx = jnp.concatenate([edge_index, etype[None]], axis=0)
    idx = idx.reshape(3, NW * NB, B)

    kv, q, s, relw, r_out = _dense_call(
        n_in_feats, Wk_w, Wk_b.reshape(1, D), Wq_w, Wq_b.reshape(1, D),
        Wv_w, Wv_b.reshape(1, D), W_S_w, W_S_b.reshape(1, D),
        w_comp, relation_att, r_feats, W_R_w, W_R_b.reshape(1, D))

    acc, den = _edge_call(kv, q, relw, idx)

    n_out = _fin_call(acc, den, s, alpha.reshape(1, 1),
                      bn_gamma.reshape(1, D), bn_beta.reshape(1, D))
    return n_out, r_out


# speedup vs baseline: 1.5084x; 1.1092x over previous
import functools

import jax
import jax.numpy as jnp
from jax import lax
from jax.experimental import pallas as pl
from jax.experimental.pallas import tpu as pltpu
from jax.experimental.pallas import tpu_sc as plsc

N_NODES = 10000
N_EDGES = 320000
D = 128
NUM_RELS = 50
BN_EPS = 1e-5

NC = 2
NS = 16
NW = NC * NS
EW = N_EDGES // NW
B = 80
NB = EW // B
AW = D + 16
NCHUNKS = N_NODES // B


def _dense_body(x_ref, wk_ref, bk_ref, wq_ref, bq_ref, wv_ref, bv_ref,
                ws_ref, bs_ref, wc_ref, ra_ref, rf_ref, wr_ref, br_ref,
                kv_ref, q_ref, s_ref, relw_ref, rout_ref):
    x = x_ref[...]

    def proj(w_ref, b_ref):
        return lax.dot_general(x, w_ref[...], (((1,), (1,)), ((), ())),
                               preferred_element_type=jnp.float32) + b_ref[...]

    def pack_halves(x):
        w = x.shape[1] // 2

        def rne(y):
            t = lax.bitcast_convert_type(y, jnp.int32)
            return t + 0x7FFF + ((t >> 16) & 1)

        lo = lax.shift_right_logical(rne(x[:, :w]), 16)
        hi = lax.bitwise_and(rne(x[:, w:]), jnp.int32(-65536))
        return lax.bitwise_or(hi, lo)

    kv_ref[:, :D // 2] = pack_halves(proj(wk_ref, bk_ref))
    kv_ref[:, D // 2:] = pack_halves(proj(wv_ref, bv_ref))
    q_ref[...] = pack_halves(proj(wq_ref, bq_ref))
    s_ref[...] = proj(ws_ref, bs_ref)
    relw_ref[...] = pack_halves(lax.dot_general(
        wc_ref[...], ra_ref[...], (((1,), (0,)), ((), ())),
        preferred_element_type=jnp.float32))
    rout_ref[...] = lax.dot_general(rf_ref[...], wr_ref[...],
                                    (((1,), (1,)), ((), ())),
                                    preferred_element_type=jnp.float32) + br_ref[...]


_dense_call = pl.pallas_call(
    _dense_body,
    out_shape=[
        jax.ShapeDtypeStruct((N_NODES, D), jnp.int32),
        jax.ShapeDtypeStruct((N_NODES, D // 2), jnp.int32),
        jax.ShapeDtypeStruct((N_NODES, D), jnp.float32),
        jax.ShapeDtypeStruct((NUM_RELS, D // 2), jnp.int32),
        jax.ShapeDtypeStruct((NUM_RELS - 1, D), jnp.float32),
    ],
)


def _edge_body(kv_hbm, q_hbm, relw_hbm, eidx_hbm, et_hbm, outv_hbm,
               outd_hbm,
               idx_v, relw_v, kv_v, q_v, pv_v, pd_v, acc_sh, den_sh,
               sem_k, sem_q, sem_i, sem_s, sem_d):
    c = lax.axis_index("c")
    s = lax.axis_index("s")
    wid = s * NC + c
    DW = D // 2

    pltpu.sync_copy(relw_hbm, relw_v)

    z16 = jnp.zeros((16,), jnp.float32)

    def zero_row(r, _):
        for cc in range(D // 16):
            pv_v[r, pl.ds(cc * 16, 16)] = z16
        pd_v[r, pl.ds(0, 16)] = z16
        return 0

    lax.fori_loop(0, B, zero_row, 0)

    for i in range(-(-NCHUNKS // NS)):
        ch = s + i * NS

        @pl.when(ch < NCHUNKS)
        def _():
            pltpu.sync_copy(pv_v, acc_sh.at[pl.ds(ch * B, B)])
            pltpu.sync_copy(pd_v, den_sh.at[pl.ds(ch * B, B)])
    plsc.subcore_barrier()

    lanes = lax.iota(jnp.int32, 16)

    def idx_src(i, batch):
        if i < 2:
            return eidx_hbm.at[i, wid * NB + batch]
        return et_hbm.at[wid * NB + batch]

    def idx_fetch(slot, batch, sync):
        for i in range(3):
            if sync:
                pltpu.sync_copy(idx_src(i, batch), idx_v.at[slot, i])
            else:
                pltpu.async_copy(idx_src(i, batch), idx_v.at[slot, i], sem_i)

    idx_fetch(0, 0, True)
    idx_fetch(1, lax.min(1, NB - 1), False)
    pltpu.async_copy(kv_hbm.at[idx_v.at[0].at[0]], kv_v, sem_k)
    pltpu.async_copy(q_hbm.at[idx_v.at[0].at[1]], q_v.at[0], sem_q)

    def batch_body(b, _):
        m = lax.rem(b, 3)
        m1 = lax.rem(b + 1, 3)
        m2 = lax.rem(b + 2, 3)
        mq = lax.rem(b, 2)
        bn = lax.min(b + 1, NB - 1)
        bn2 = lax.min(b + 2, NB - 1)
        cur = idx_v.at[m]
        nxt = idx_v.at[m1]
        with jax.named_scope("ph_gwait"):
            pltpu.make_async_copy(kv_hbm.at[cur.at[0]], kv_v, sem_k).wait()
            pltpu.make_async_copy(q_hbm.at[cur.at[1]], q_v.at[mq],
                                  sem_q).wait()

            @pl.when(b > 0)
            def _():
                pltpu.make_async_copy(pd_v, den_sh.at[cur.at[1]],
                                      sem_d).wait()
        with jax.named_scope("ph_qissue"):
            for i in range(3):
                pltpu.make_async_copy(idx_src(i, bn), idx_v.at[m1, i],
                                      sem_i).wait()
            pltpu.async_copy(q_hbm.at[nxt.at[1]], q_v.at[1 - mq], sem_q)
        ns_att = jax.named_scope("ph_att"); ns_att.__enter__()
        for g in range(B // 16):
            row = lanes + (g * 16)
            et = idx_v[m, 2, pl.ds(g * 16, 16)]

            def att_body(j, acc4):
                accs = list(acc4)
                for t in range(2):
                    jc = (jnp.full((16,), j, jnp.int32) + t + lanes) & (DW - 1)
                    kk = plsc.bitcast(plsc.load_gather(kv_v, [row, jc]),
                                      jnp.bfloat16)
                    ww = plsc.bitcast(plsc.load_gather(relw_v, [et, jc]),
                                      jnp.bfloat16)
                    qq = plsc.bitcast(
                        plsc.load_gather(q_v.at[mq], [row, jc]),
                        jnp.bfloat16)
                    lo, hi = plsc.unpack(kk * ww * qq,
                                         format=plsc.PackFormat.INTERLEAVED)
                    accs[2 * t] = accs[2 * t] + lo
                    accs[2 * t + 1] = accs[2 * t + 1] + hi
                return tuple(accs)

            z = jnp.zeros((16,), jnp.float32)
            a0, a1, a2, a3 = plsc.parallel_loop(
                0, DW, 2, unroll=4, carry=(z, z, z, z))(att_body)
            att = (a0 + a1) + (a2 + a3)
            plsc.store_scatter(pd_v, [row, lanes], jnp.exp(att))
        ns_att.__exit__(None, None, None)
        with jax.named_scope("ph_issue"):
            pltpu.async_copy(kv_hbm.at[nxt.at[0]], kv_v, sem_k)
        with jax.named_scope("ph_vwait"):
            @pl.when(b > 0)
            def _():
                pltpu.make_async_copy(pv_v, acc_sh.at[cur.at[1]],
                                      sem_s).wait()
        ns_pv = jax.named_scope("ph_pv"); ns_pv.__enter__()
        for g in range(B // 16):
            row = lanes + (g * 16)
            p = plsc.load_gather(pd_v, [row, lanes])

            @plsc.parallel_loop(0, DW, 2, unroll=2)
            def pv_body(j):
                for t in range(2):
                    jc = (jnp.full((16,), j, jnp.int32) + t + lanes) & (DW - 1)
                    vv = plsc.bitcast(plsc.load_gather(kv_v, [row, jc + DW]),
                                      jnp.bfloat16)
                    lo, hi = plsc.unpack(vv,
                                         format=plsc.PackFormat.INTERLEAVED)
                    plsc.store_scatter(pv_v, [row, jc], p * lo)
                    plsc.store_scatter(pv_v, [row, jc + DW], p * hi)
        ns_pv.__exit__(None, None, None)
        with jax.named_scope("ph_scat"):
            pltpu.async_copy(pv_v, acc_sh.at[cur.at[1]], sem_s, add=True)
            pltpu.async_copy(pd_v, den_sh.at[cur.at[1]], sem_d, add=True)
        with jax.named_scope("ph_ipf"):
            idx_fetch(m2, bn2, False)
        return 0

    lax.fori_loop(0, NB, batch_body, 0)
    pltpu.make_async_copy(kv_hbm.at[idx_v.at[0].at[0]], kv_v, sem_k).wait()
    pltpu.make_async_copy(q_hbm.at[idx_v.at[0].at[1]], q_v.at[0],
                          sem_q).wait()
    for i in range(3):
        pltpu.make_async_copy(idx_src(i, 0), idx_v.at[0, i], sem_i).wait()
    pltpu.make_async_copy(pv_v, acc_sh.at[idx_v.at[0].at[1]], sem_s).wait()
    pltpu.make_async_copy(pd_v, den_sh.at[idx_v.at[0].at[1]], sem_d).wait()
    plsc.subcore_barrier()

    for i in range(-(-NCHUNKS // NS)):
        ch = s + i * NS

        @pl.when(ch < NCHUNKS)
        def _():
            pltpu.sync_copy(acc_sh.at[pl.ds(ch * B, B)],
                            outv_hbm.at[c, pl.ds(ch * B, B)])
            pltpu.sync_copy(den_sh.at[pl.ds(ch * B, B)],
                            outd_hbm.at[c, pl.ds(ch * B, B)])


_edge_call = functools.partial(
    pl.kernel,
    out_type=[
        jax.ShapeDtypeStruct((NC, N_NODES, D), jnp.float32),
        jax.ShapeDtypeStruct((NC, N_NODES, 16), jnp.float32),
    ],
    mesh=plsc.VectorSubcoreMesh(core_axis_name="c", subcore_axis_name="s",
                                num_cores=NC, num_subcores=NS),
    compiler_params=pltpu.CompilerParams(use_tc_tiling_on_sc=False,
                                         needs_layout_passes=False),
    scratch_types=[
        pltpu.VMEM((3, 3, B), jnp.int32),
        pltpu.VMEM((NUM_RELS, D // 2), jnp.int32),
        pltpu.VMEM((B, D), jnp.int32),
        pltpu.VMEM((2, B, D // 2), jnp.int32),
        pltpu.VMEM((B, D), jnp.float32),
        pltpu.VMEM((B, 16), jnp.float32),
        pltpu.VMEM_SHARED((N_NODES, D), jnp.float32),
        pltpu.VMEM_SHARED((N_NODES, 16), jnp.float32),
        pltpu.SemaphoreType.DMA,
        pltpu.SemaphoreType.DMA,
        pltpu.SemaphoreType.DMA,
        pltpu.SemaphoreType.DMA,
        pltpu.SemaphoreType.DMA,
    ],
)(_edge_body)


def _fin_body(acc_ref, den_ref, s_ref, alpha_ref, g_ref, b_ref, out_ref):
    num = acc_ref[0] + acc_ref[1]
    den = jnp.sum(den_ref[0] + den_ref[1], axis=1, keepdims=True)
    final = num / jnp.where(den > 0.0, den, 1.0)
    a = jax.nn.sigmoid(alpha_ref[0, 0])
    n_out = a * s_ref[...] + (1.0 - a) * final
    mean = jnp.mean(n_out, axis=0, keepdims=True)
    var = jnp.mean((n_out - mean) ** 2, axis=0, keepdims=True)
    out_ref[...] = jnp.tanh((n_out - mean) * lax.rsqrt(var + BN_EPS)
                            * g_ref[...] + b_ref[...])


_fin_call = pl.pallas_call(
    _fin_body,
    out_shape=jax.ShapeDtypeStruct((N_NODES, D), jnp.float32),
)


def kernel(n_in_feats, r_feats, edge_index, etype, norm,
           W_S_w, W_S_b, Wk_w, Wk_b, Wq_w, Wq_b, Wv_w, Wv_b,
           W_R_w, W_R_b, relation_att, w_comp, alpha, loop_rel,
           bn_gamma, bn_beta):
    del norm, loop_rel
    eidx = edge_index.reshape(2, NW * NB, B)
    et = etype.reshape(NW * NB, B)

    kv, q, s, relw, r_out = _dense_call(
        n_in_feats, Wk_w, Wk_b.reshape(1, D), Wq_w, Wq_b.reshape(1, D),
        Wv_w, Wv_b.reshape(1, D), W_S_w, W_S_b.reshape(1, D),
        w_comp, relation_att, r_feats, W_R_w, W_R_b.reshape(1, D))

    acc, den = _edge_call(kv, q, relw, eidx, et)

    n_out = _fin_call(acc, den, s, alpha.reshape(1, 1),
                      bn_gamma.reshape(1, D), bn_beta.reshape(1, D))
    return n_out, r_out
